# Initial kernel scaffold; baseline (speedup 1.0000x reference)
import jax
import jax.numpy as jnp
from jax.experimental import pallas as pl


def kernel(positions, edge_index, Wq, bq, Wk, bk, Wv, bv, Wo, bo, gamma, beta):
    raise NotImplementedError("write your pallas kernel here")


# speedup vs baseline: 8.4141x; 8.4141x over previous
import jax
import jax.numpy as jnp
import numpy as np
from jax import lax
from jax.experimental import pallas as pl
from jax.experimental.pallas import tpu as pltpu
from jax.experimental.pallas import tpu_sc as plsc

_N = 50000
_E = 1600000
_HD = 8
_NH = 4
_D = _HD * _NH

_NC = 2
_NS = 16
_L = 16
_NW = _NC * _NS

_CH = 128
_NROWS = _E // _CH
_ROWS_BASE = _NROWS // _NW
_ROWS_EXTRA = _NROWS % _NW

_NP = 51200
_PER_SUB = _NP // _NS

_OFF_M = 0
_OFF_C = 40
_OFF_D = 136
_NCONST = 168


def _sc_edge_kernel(pos_hbm, eidx_hbm, consts_hbm, zero1_hbm,
                    out_sums, out_counts,
                    consts_v, ridx, cidx, st_v, ones_v,
                    gxr, gyr, gzr, gxc, gyc, gzc,
                    ob0, ob1, ob2, ob3, ob4, ob5, ob6, ob7,
                    px_sp, py_sp, pz_sp,
                    a0, a1, a2, a3, a4, a5, a6, a7, cnt_sp, sem):
    c = lax.axis_index("c")
    s = lax.axis_index("s")
    wid = s * _NC + c
    obs = (ob0, ob1, ob2, ob3, ob4, ob5, ob6, ob7)
    accs = (a0, a1, a2, a3, a4, a5, a6, a7)
    psp = (px_sp, py_sp, pz_sp)

    pltpu.sync_copy(consts_hbm, consts_v)

    nbase = s * _PER_SUB

    def _stage(i, _):
        csl = pl.ds(nbase + i * _CH, _CH)
        for d in range(3):
            pltpu.sync_copy(pos_hbm.at[pl.ds(d * _NP + nbase + i * _CH, _CH)],
                            st_v)
            pltpu.sync_copy(st_v, psp[d].at[csl])
        pltpu.sync_copy(zero1_hbm.at[csl], st_v)
        for j in range(_HD):
            pltpu.sync_copy(st_v, accs[j].at[csl])
        pltpu.sync_copy(st_v, cnt_sp.at[csl])
        return _
    lax.fori_loop(0, _PER_SUB // _CH, _stage, None)

    def _fill_ones(i, _):
        ones_v[pl.ds(i * _L, _L)] = jnp.full((_L,), 1.0, jnp.float32)
        return _
    lax.fori_loop(0, _CH // _L, _fill_ones, None)

    plsc.subcore_barrier()

    base_row = wid * _ROWS_BASE + jnp.minimum(wid, _ROWS_EXTRA)
    nrows = _ROWS_BASE + jnp.where(wid < _ROWS_EXTRA, 1, 0)

    def cv(k):
        return consts_v[pl.ds(k * _L, _L)]

    def _row_body(i, _):
        r = base_row + i
        pltpu.sync_copy(eidx_hbm.at[0, r, 0], ridx.at[0])
        pltpu.sync_copy(eidx_hbm.at[1, r, 0], cidx.at[0])
        pltpu.async_copy(px_sp.at[ridx.at[0]], gxr, sem).wait()
        pltpu.async_copy(py_sp.at[ridx.at[0]], gyr, sem).wait()
        pltpu.async_copy(pz_sp.at[ridx.at[0]], gzr, sem).wait()
        pltpu.async_copy(px_sp.at[cidx.at[0]], gxc, sem).wait()
        pltpu.async_copy(py_sp.at[cidx.at[0]], gyc, sem).wait()
        pltpu.async_copy(pz_sp.at[cidx.at[0]], gzc, sem).wait()

        def _group(g, _c):
            gs = pl.ds(g * _L, _L)
            x = gxr[gs] - gxc[gs]
            y = gyr[gs] - gyc[gs]
            z = gzr[gs] - gzc[gs]
            feats = (x * x, y * y, z * z, x * y, x * z, y * z, x, y, z)

            logits = []
            for h in range(_NH):
                acc = cv(_OFF_M + 9 * 4 + h)
                for f in range(9):
                    acc = acc + feats[f] * cv(_OFF_M + f * 4 + h)
                logits.append(acc)
            m = jnp.maximum(jnp.maximum(logits[0], logits[1]),
                            jnp.maximum(logits[2], logits[3]))
            ex = [jnp.exp(l - m) for l in logits]
            inv = 1.0 / (ex[0] + ex[1] + ex[2] + ex[3])

            for j in range(_HD):
                accj = None
                for h in range(_NH):
                    u = cv(_OFF_D + h * _HD + j)
                    u = u + x * cv(_OFF_C + h * 24 + 0 * _HD + j)
                    u = u + y * cv(_OFF_C + h * 24 + 1 * _HD + j)
                    u = u + z * cv(_OFF_C + h * 24 + 2 * _HD + j)
                    t = ex[h] * u
                    accj = t if accj is None else accj + t
                obs[j][gs] = accj * inv
            return _c
        lax.fori_loop(0, _CH // _L, _group, None)

        for j in range(_HD):
            pltpu.sync_copy(obs[j], accs[j].at[cidx.at[0]], add=True)
        pltpu.sync_copy(ones_v, cnt_sp.at[cidx.at[0]], add=True)
        return _
    lax.fori_loop(0, nrows, _row_body, None)

    plsc.subcore_barrier()

    def _writeback(i, _):
        csl = pl.ds(nbase + i * _CH, _CH)
        for j in range(_HD):
            pltpu.sync_copy(accs[j].at[csl], st_v)
            pltpu.sync_copy(
                st_v,
                out_sums.at[pl.ds((j * _NC + c) * _NP + nbase + i * _CH,
                                  _CH)])
        pltpu.sync_copy(cnt_sp.at[csl], st_v)
        pltpu.sync_copy(st_v,
                        out_counts.at[pl.ds(c * _NP + nbase + i * _CH, _CH)])
        return _
    lax.fori_loop(0, _PER_SUB // _CH, _writeback, None)


def _tc_epilogue(sums_ref, counts_ref, bo_ref, gamma_ref, beta_ref, out_ref):
    o = sums_ref[:, 0, :] + sums_ref[:, 1, :]
    cnt = jnp.sum(counts_ref[...], axis=0, keepdims=True)
    o = o / jnp.maximum(cnt, 1.0) + bo_ref[...]
    mu = jnp.mean(o, axis=0, keepdims=True)
    var = jnp.mean((o - mu) ** 2, axis=0, keepdims=True)
    o = (o - mu) * lax.rsqrt(var + 1e-5) * gamma_ref[...] + beta_ref[...]
    out_ref[...] = o * (1.0 / (1.0 + jnp.exp(-o)))


def kernel(positions, edge_index, Wq, bq, Wk, bk, Wv, bv, Wo, bo, gamma, beta):
    scale = 1.0 / np.sqrt(_HD)
    Wq3 = Wq.reshape(3, _NH, _HD)
    Wk3 = Wk.reshape(3, _NH, _HD)
    Wv3 = Wv.reshape(3, _NH, _HD)
    bq2 = bq.reshape(_NH, _HD)
    bk2 = bk.reshape(_NH, _HD)
    bv2 = bv.reshape(_NH, _HD)
    Wo3 = Wo.reshape(_NH, _HD, _HD)
    A = jnp.einsum('ihd,jhd->hij', Wq3, Wk3) * scale
    lin = (jnp.einsum('ihd,hd->hi', Wq3, bk2)
           + jnp.einsum('ihd,hd->hi', Wk3, bq2)) * scale
    cst = (bq2 * bk2).sum(-1) * scale
    C = jnp.einsum('ihd,hdj->hij', Wv3, Wo3)
    dv = jnp.einsum('hd,hdj->hj', bv2, Wo3)
    M = jnp.stack([A[:, 0, 0], A[:, 1, 1], A[:, 2, 2],
                   A[:, 0, 1] + A[:, 1, 0], A[:, 0, 2] + A[:, 2, 0],
                   A[:, 1, 2] + A[:, 2, 1],
                   lin[:, 0], lin[:, 1], lin[:, 2], cst], axis=0)
    const_flat = jnp.concatenate(
        [M.reshape(-1), C.reshape(-1), dv.reshape(-1)])
    consts = jnp.repeat(const_flat, _L)

    posT = jnp.pad(positions.T, ((0, 0), (0, _NP - _N)))
    pos_flat = posT.reshape(-1)
    eidx4 = edge_index.reshape(2, _NROWS, 1, _CH)
    zero1 = jnp.zeros((_NP,), jnp.float32)

    mesh = plsc.VectorSubcoreMesh(core_axis_name="c", subcore_axis_name="s",
                                  num_cores=_NC, num_subcores=_NS)
    scr = [pltpu.VMEM((_NCONST * _L,), jnp.float32)]
    scr += [pltpu.VMEM((1, _CH), jnp.int32)] * 2
    scr += [pltpu.VMEM((_CH,), jnp.float32)] * 2
    scr += [pltpu.VMEM((_CH,), jnp.float32)] * 6
    scr += [pltpu.VMEM((_CH,), jnp.float32)] * _HD
    scr += [pltpu.VMEM_SHARED((_NP,), jnp.float32)] * 3
    scr += [pltpu.VMEM_SHARED((_NP,), jnp.float32)] * (_HD + 1)
    scr += [pltpu.SemaphoreType.DMA]
    sc_call = pl.kernel(
        _sc_edge_kernel,
        out_type=[jax.ShapeDtypeStruct((_HD * _NC * _NP,), jnp.float32),
                  jax.ShapeDtypeStruct((_NC * _NP,), jnp.float32)],
        mesh=mesh,
        scratch_types=scr,
    )
    sums_flat, counts_flat = sc_call(pos_flat, eidx4, consts, zero1)

    sums3 = sums_flat.reshape(_HD, _NC, _NP)
    counts2 = counts_flat.reshape(_NC, _NP)
    B = 3200
    grid = _NP // B
    outT = pl.pallas_call(
        _tc_epilogue,
        out_shape=jax.ShapeDtypeStruct((_HD, _NP), jnp.float32),
        grid=(grid,),
        in_specs=[
            pl.BlockSpec((_HD, _NC, B), lambda i: (0, 0, i)),
            pl.BlockSpec((_NC, B), lambda i: (0, i)),
            pl.BlockSpec((_HD, 1), lambda i: (0, 0)),
            pl.BlockSpec((_HD, 1), lambda i: (0, 0)),
            pl.BlockSpec((_HD, 1), lambda i: (0, 0)),
        ],
        out_specs=pl.BlockSpec((_HD, B), lambda i: (0, i)),
    )(sums3, counts2, bo.reshape(_HD, 1), gamma.reshape(_HD, 1),
      beta.reshape(_HD, 1))
    return outT.T[:_N]


# speedup vs baseline: 12.1826x; 1.4479x over previous
import jax
import jax.numpy as jnp
import numpy as np
from jax import lax
from jax.experimental import pallas as pl
from jax.experimental.pallas import tpu as pltpu
from jax.experimental.pallas import tpu_sc as plsc

_N = 50000
_E = 1600000
_HD = 8
_NH = 4
_D = _HD * _NH

_NC = 2
_NS = 16
_L = 16
_NW = _NC * _NS

_CH = 128
_NROWS = _E // _CH
_ROWS_BASE = _NROWS // _NW
_ROWS_EXTRA = _NROWS % _NW

_NP = 51200
_PER_SUB = _NP // _NS

_OFF_M = 0
_OFF_C = 40
_OFF_D = 136
_NCONST = 168


def _sc_edge_kernel(pos_hbm, eidx_hbm, consts_hbm, zero1_hbm,
                    out_sums, out_counts,
                    consts_v, ridx, cidx, st_v, ones_v,
                    gxr, gyr, gzr, gxc, gyc, gzc,
                    ob0, ob1, ob2, ob3, ob4, ob5, ob6, ob7,
                    px_sp, py_sp, pz_sp,
                    a0, a1, a2, a3, a4, a5, a6, a7, cnt_sp, sem):
    c = lax.axis_index("c")
    s = lax.axis_index("s")
    wid = s * _NC + c
    obs = (ob0, ob1, ob2, ob3, ob4, ob5, ob6, ob7)
    accs = (a0, a1, a2, a3, a4, a5, a6, a7)
    psp = (px_sp, py_sp, pz_sp)

    pltpu.sync_copy(consts_hbm, consts_v)

    nbase = s * _PER_SUB

    def _stage(i, _):
        csl = pl.ds(nbase + i * _CH, _CH)
        for d in range(3):
            pltpu.sync_copy(pos_hbm.at[pl.ds(d * _NP + nbase + i * _CH, _CH)],
                            st_v)
            pltpu.sync_copy(st_v, psp[d].at[csl])
        pltpu.sync_copy(zero1_hbm.at[csl], st_v)
        for j in range(_HD):
            pltpu.sync_copy(st_v, accs[j].at[csl])
        pltpu.sync_copy(st_v, cnt_sp.at[csl])
        return _
    lax.fori_loop(0, _PER_SUB // _CH, _stage, None)

    def _fill_ones(i, _):
        ones_v[pl.ds(i * _L, _L)] = jnp.full((_L,), 1.0, jnp.float32)
        return _
    lax.fori_loop(0, _CH // _L, _fill_ones, None)

    plsc.subcore_barrier()

    base_row = wid * _ROWS_BASE + jnp.minimum(wid, _ROWS_EXTRA)
    nrows = _ROWS_BASE + jnp.where(wid < _ROWS_EXTRA, 1, 0)

    def cv(k):
        return consts_v[pl.ds(k * _L, _L)]

    def _row_body(i, _):
        r = base_row + i
        d0 = pltpu.async_copy(eidx_hbm.at[0, r, 0], ridx.at[0], sem)
        d1 = pltpu.async_copy(eidx_hbm.at[1, r, 0], cidx.at[0], sem)
        d0.wait()
        d1.wait()
        descs = [pltpu.async_copy(px_sp.at[ridx.at[0]], gxr, sem),
                 pltpu.async_copy(py_sp.at[ridx.at[0]], gyr, sem),
                 pltpu.async_copy(pz_sp.at[ridx.at[0]], gzr, sem),
                 pltpu.async_copy(px_sp.at[cidx.at[0]], gxc, sem),
                 pltpu.async_copy(py_sp.at[cidx.at[0]], gyc, sem),
                 pltpu.async_copy(pz_sp.at[cidx.at[0]], gzc, sem)]
        for d in descs:
            d.wait()

        def _group(g, _c):
            gs = pl.ds(g * _L, _L)
            x = gxr[gs] - gxc[gs]
            y = gyr[gs] - gyc[gs]
            z = gzr[gs] - gzc[gs]
            feats = (x * x, y * y, z * z, x * y, x * z, y * z, x, y, z)

            logits = []
            for h in range(_NH):
                acc = cv(_OFF_M + 9 * 4 + h)
                for f in range(9):
                    acc = acc + feats[f] * cv(_OFF_M + f * 4 + h)
                logits.append(acc)
            m = jnp.maximum(jnp.maximum(logits[0], logits[1]),
                            jnp.maximum(logits[2], logits[3]))
            ex = [jnp.exp(l - m) for l in logits]
            inv = 1.0 / (ex[0] + ex[1] + ex[2] + ex[3])

            for j in range(_HD):
                accj = None
                for h in range(_NH):
                    u = cv(_OFF_D + h * _HD + j)
                    u = u + x * cv(_OFF_C + h * 24 + 0 * _HD + j)
                    u = u + y * cv(_OFF_C + h * 24 + 1 * _HD + j)
                    u = u + z * cv(_OFF_C + h * 24 + 2 * _HD + j)
                    t = ex[h] * u
                    accj = t if accj is None else accj + t
                obs[j][gs] = accj * inv
            return _c
        lax.fori_loop(0, _CH // _L, _group, None)

        sdescs = [pltpu.async_copy(obs[j], accs[j].at[cidx.at[0]], sem,
                                   add=True)
                  for j in range(_HD)]
        sdescs.append(pltpu.async_copy(ones_v, cnt_sp.at[cidx.at[0]], sem,
                                       add=True))
        for d in sdescs:
            d.wait()
        return _
    lax.fori_loop(0, nrows, _row_body, None)

    plsc.subcore_barrier()

    def _writeback(i, _):
        csl = pl.ds(nbase + i * _CH, _CH)
        for j in range(_HD):
            pltpu.sync_copy(accs[j].at[csl], st_v)
            pltpu.sync_copy(
                st_v,
                out_sums.at[pl.ds((j * _NC + c) * _NP + nbase + i * _CH,
                                  _CH)])
        pltpu.sync_copy(cnt_sp.at[csl], st_v)
        pltpu.sync_copy(st_v,
                        out_counts.at[pl.ds(c * _NP + nbase + i * _CH, _CH)])
        return _
    lax.fori_loop(0, _PER_SUB // _CH, _writeback, None)


def _tc_epilogue(sums_ref, counts_ref, bo_ref, gamma_ref, beta_ref, out_ref):
    o = sums_ref[:, 0, :] + sums_ref[:, 1, :]
    cnt = jnp.sum(counts_ref[...], axis=0, keepdims=True)
    o = o / jnp.maximum(cnt, 1.0) + bo_ref[...]
    mu = jnp.mean(o, axis=0, keepdims=True)
    var = jnp.mean((o - mu) ** 2, axis=0, keepdims=True)
    o = (o - mu) * lax.rsqrt(var + 1e-5) * gamma_ref[...] + beta_ref[...]
    out_ref[...] = o * (1.0 / (1.0 + jnp.exp(-o)))


def kernel(positions, edge_index, Wq, bq, Wk, bk, Wv, bv, Wo, bo, gamma, beta):
    scale = 1.0 / np.sqrt(_HD)
    Wq3 = Wq.reshape(3, _NH, _HD)
    Wk3 = Wk.reshape(3, _NH, _HD)
    Wv3 = Wv.reshape(3, _NH, _HD)
    bq2 = bq.reshape(_NH, _HD)
    bk2 = bk.reshape(_NH, _HD)
    bv2 = bv.reshape(_NH, _HD)
    Wo3 = Wo.reshape(_NH, _HD, _HD)
    A = jnp.einsum('ihd,jhd->hij', Wq3, Wk3) * scale
    lin = (jnp.einsum('ihd,hd->hi', Wq3, bk2)
           + jnp.einsum('ihd,hd->hi', Wk3, bq2)) * scale
    cst = (bq2 * bk2).sum(-1) * scale
    C = jnp.einsum('ihd,hdj->hij', Wv3, Wo3)
    dv = jnp.einsum('hd,hdj->hj', bv2, Wo3)
    M = jnp.stack([A[:, 0, 0], A[:, 1, 1], A[:, 2, 2],
                   A[:, 0, 1] + A[:, 1, 0], A[:, 0, 2] + A[:, 2, 0],
                   A[:, 1, 2] + A[:, 2, 1],
                   lin[:, 0], lin[:, 1], lin[:, 2], cst], axis=0)
    const_flat = jnp.concatenate(
        [M.reshape(-1), C.reshape(-1), dv.reshape(-1)])
    consts = jnp.repeat(const_flat, _L)

    posT = jnp.pad(positions.T, ((0, 0), (0, _NP - _N)))
    pos_flat = posT.reshape(-1)
    eidx4 = edge_index.reshape(2, _NROWS, 1, _CH)
    zero1 = jnp.zeros((_NP,), jnp.float32)

    mesh = plsc.VectorSubcoreMesh(core_axis_name="c", subcore_axis_name="s",
                                  num_cores=_NC, num_subcores=_NS)
    scr = [pltpu.VMEM((_NCONST * _L,), jnp.float32)]
    scr += [pltpu.VMEM((1, _CH), jnp.int32)] * 2
    scr += [pltpu.VMEM((_CH,), jnp.float32)] * 2
    scr += [pltpu.VMEM((_CH,), jnp.float32)] * 6
    scr += [pltpu.VMEM((_CH,), jnp.float32)] * _HD
    scr += [pltpu.VMEM_SHARED((_NP,), jnp.float32)] * 3
    scr += [pltpu.VMEM_SHARED((_NP,), jnp.float32)] * (_HD + 1)
    scr += [pltpu.SemaphoreType.DMA]
    sc_call = pl.kernel(
        _sc_edge_kernel,
        out_type=[jax.ShapeDtypeStruct((_HD * _NC * _NP,), jnp.float32),
                  jax.ShapeDtypeStruct((_NC * _NP,), jnp.float32)],
        mesh=mesh,
        scratch_types=scr,
    )
    sums_flat, counts_flat = sc_call(pos_flat, eidx4, consts, zero1)

    sums3 = sums_flat.reshape(_HD, _NC, _NP)
    counts2 = counts_flat.reshape(_NC, _NP)
    B = 3200
    grid = _NP // B
    outT = pl.pallas_call(
        _tc_epilogue,
        out_shape=jax.ShapeDtypeStruct((_HD, _NP), jnp.float32),
        grid=(grid,),
        in_specs=[
            pl.BlockSpec((_HD, _NC, B), lambda i: (0, 0, i)),
            pl.BlockSpec((_NC, B), lambda i: (0, i)),
            pl.BlockSpec((_HD, 1), lambda i: (0, 0)),
            pl.BlockSpec((_HD, 1), lambda i: (0, 0)),
            pl.BlockSpec((_HD, 1), lambda i: (0, 0)),
        ],
        out_specs=pl.BlockSpec((_HD, B), lambda i: (0, i)),
    )(sums3, counts2, bo.reshape(_HD, 1), gamma.reshape(_HD, 1),
      beta.reshape(_HD, 1))
    return outT.T[:_N]
